# baseline (speedup 1.0000x reference)
import functools

import jax
import jax.numpy as jnp
from jax import lax
from jax.experimental import pallas as pl
from jax.experimental.pallas import tpu as pltpu
from jax.experimental.pallas import tpu_sc as plsc

EMBED_DIM = 64
BATCH = 16384


@functools.cache
def _build():
    info = plsc.get_sparse_core_info()
    NC, NS = info.num_cores, info.num_subcores
    NW = NC * NS
    b_per_w = BATCH // NW
    mesh = plsc.VectorSubcoreMesh(core_axis_name="c", subcore_axis_name="s")

    out_t = jax.ShapeDtypeStruct((BATCH, EMBED_DIM), jnp.float32)

    @functools.partial(
        pl.kernel,
        mesh=mesh,
        out_type=out_t,
        compiler_params=pltpu.CompilerParams(use_tc_tiling_on_sc=False),
        scratch_types=[
            pltpu.VMEM((b_per_w,), jnp.int32),
            pltpu.VMEM((b_per_w, EMBED_DIM), jnp.float32),
            pltpu.SemaphoreType.DMA,
        ],
    )
    def lookup(W2, ids, out, idx_v, buf, sem):
        wid = lax.axis_index("s") * NC + lax.axis_index("c")
        base = wid * b_per_w

        pltpu.sync_copy(ids.at[pl.ds(base, b_per_w)], idx_v)
        pltpu.async_copy(W2.at[idx_v], buf, sem).wait()
        pltpu.sync_copy(buf, out.at[pl.ds(base, b_per_w)])

    return lookup


def kernel(W_user, W_item, W_category, user_id, item_id, category_id):
    lookup = _build()
    e_user = lookup(W_user, user_id.astype(jnp.int32))
    e_item = lookup(W_item, item_id.astype(jnp.int32))
    e_category = lookup(W_category, category_id.astype(jnp.int32))
    return (e_user, e_item, e_category)


# speedup vs baseline: 1.9931x; 1.9931x over previous
import functools

import jax
import jax.numpy as jnp
from jax import lax
from jax.experimental import pallas as pl
from jax.experimental.pallas import tpu as pltpu
from jax.experimental.pallas import tpu_sc as plsc

EMBED_DIM = 64
BATCH = 16384
CHUNK = 32


@functools.cache
def _build():
    info = plsc.get_sparse_core_info()
    NC, NS = info.num_cores, info.num_subcores
    NW = NC * NS
    b_per_w = BATCH // NW
    n_chunks = b_per_w // CHUNK
    mesh = plsc.VectorSubcoreMesh(core_axis_name="c", subcore_axis_name="s")

    out_t = jax.ShapeDtypeStruct((BATCH, EMBED_DIM), jnp.float32)

    @functools.partial(
        pl.kernel,
        mesh=mesh,
        out_type=[out_t, out_t, out_t],
        scratch_types=[
            pltpu.VMEM((b_per_w,), jnp.int32),
            pltpu.VMEM((CHUNK, 8, EMBED_DIM), jnp.float32),
            pltpu.VMEM((CHUNK, 8, EMBED_DIM), jnp.float32),
            pltpu.VMEM((CHUNK, EMBED_DIM), jnp.float32),
            pltpu.SemaphoreType.DMA,
            pltpu.SemaphoreType.DMA,
        ],
    )
    def lookup(W_u, W_i, W_c, id_u, id_i, id_c, out_u, out_i, out_c,
               idx_v, tiles0, tiles1, obuf, sem0, sem1):
        wid = lax.axis_index("s") * NC + lax.axis_index("c")
        base = wid * b_per_w
        bufs = (tiles0, tiles1)
        sems = (sem0, sem1)

        for W3, ids, out in ((W_u, id_u, out_u),
                             (W_i, id_i, out_i),
                             (W_c, id_c, out_c)):
            pltpu.sync_copy(ids.at[pl.ds(base, b_per_w)], idx_v)

            def fire(c, buf, sem, W3=W3):
                for g in range(CHUNK // 16):
                    v = idx_v[pl.ds(c * CHUNK + g * 16, 16)]
                    for l in range(16):
                        t = lax.shift_right_logical(v[l], 3)
                        pltpu.async_copy(W3.at[t], buf.at[g * 16 + l], sem)

            def extract_out(c, buf, W3=W3, out=out):
                for g in range(CHUNK // 16):
                    rv = lax.bitwise_and(
                        idx_v[pl.ds(c * CHUNK + g * 16, 16)], 7
                    )
                    for l in range(16):
                        r = rv[l]
                        for k in range(EMBED_DIM // 16):
                            obuf[g * 16 + l, pl.ds(16 * k, 16)] = (
                                buf[g * 16 + l, r, pl.ds(16 * k, 16)]
                            )
                pltpu.sync_copy(obuf, out.at[pl.ds(base + c * CHUNK, CHUNK)])

            def wait(buf, sem, W3=W3):
                pltpu.make_async_copy(
                    W3.at[pl.ds(0, CHUNK)], buf, sem
                ).wait()

            fire(0, bufs[0], sems[0])

            def step(c, carry, W3=W3, out=out):
                parity = lax.rem(c, 2)

                @pl.when(parity == 0)
                def _even():
                    fire(c + 1, bufs[1], sems[1])
                    wait(bufs[0], sems[0])
                    extract_out(c, bufs[0])

                @pl.when(parity == 1)
                def _odd():
                    fire(c + 1, bufs[0], sems[0])
                    wait(bufs[1], sems[1])
                    extract_out(c, bufs[1])

                return carry

            lax.fori_loop(0, n_chunks - 1, step, 0)

            last = n_chunks - 1
            lp = (n_chunks - 1) % 2
            wait(bufs[lp], sems[lp])
            extract_out(last, bufs[lp])

    return lookup


def kernel(W_user, W_item, W_category, user_id, item_id, category_id):
    lookup = _build()
    e_user, e_item, e_category = lookup(
        W_user.reshape(-1, 8, EMBED_DIM),
        W_item.reshape(-1, 8, EMBED_DIM),
        W_category.reshape(-1, 8, EMBED_DIM),
        user_id.astype(jnp.int32),
        item_id.astype(jnp.int32),
        category_id.astype(jnp.int32),
    )
    return (e_user, e_item, e_category)
